# baseline (speedup 1.0000x reference)
import functools

import jax
import jax.numpy as jnp
from jax import lax
from jax.experimental import pallas as pl
from jax.experimental.pallas import tpu as pltpu
from jax.experimental.pallas import tpu_sc as plsc

N_SITES = 100000
K = 64
B = 16384
NC, NS, L = 2, 16, 16
NW = NC * NS
BPW = B // NW
CHUNK = 128
NCH = BPW // CHUNK

_C0 = 3.4730087503653806e-06
_C1 = 0.9997931695179835
_C2 = -0.4969826978177418
_C3 = 0.3146040987042588
_C4 = -0.18880110398530178
_C5 = 0.0817383926631096
_C6 = -0.017210668570108372

_LOG2E = 1.4426950408889634
_MAGIC = 12582912.0
_E0 = 1.0000000523545023
_E1 = 0.6931272466723738
_E2 = 0.24022211306668645
_E3 = 0.05587568397212921
_E4 = 0.009670788947925736


def _exp2_accurate(z):
    n = (z + _MAGIC) - _MAGIC
    f = z - n
    p = _E3 + f * _E4
    p = _E2 + f * p
    p = _E1 + f * p
    p = _E0 + f * p
    n_i = n.astype(jnp.int32)
    scale = lax.bitcast_convert_type(
        lax.shift_left(n_i + 127, jnp.int32(23)), jnp.float32
    )
    return p * scale


def _softplus16(x):
    z = jnp.maximum(-jnp.abs(x) * _LOG2E, -125.0)
    t = _exp2_accurate(z)
    p = _C5 + t * _C6
    p = _C4 + t * p
    p = _C3 + t * p
    p = _C2 + t * p
    p = _C1 + t * p
    p = _C0 + t * p
    return jnp.maximum(x, 0.0) + p


@functools.cache
def _get_mesh():
    return plsc.VectorSubcoreMesh(
        core_axis_name="c", subcore_axis_name="s", num_cores=NC, num_subcores=NS
    )


def _hbb_body(idx_hbm, qa_hbm, qb_hbm, out_hbm, idx_v, ra_v, rb_v, sem):
    wid = lax.axis_index("s") * NC + lax.axis_index("c")
    base = wid * NCH

    pltpu.sync_copy(idx_hbm.at[pl.ds(base, NCH)], idx_v)

    copies = []
    for j in range(NCH):
        copies.append(
            pltpu.async_copy(
                qa_hbm.at[idx_v.at[j]], ra_v.at[pl.ds(j * CHUNK, CHUNK)], sem
            )
        )
        copies.append(
            pltpu.async_copy(
                qb_hbm.at[idx_v.at[j]], rb_v.at[pl.ds(j * CHUNK, CHUNK)], sem
            )
        )
    for c in copies:
        c.wait()

    def body(r, carry):
        for c in range(K // L):
            sl = pl.ds(c * L, L)
            a = _softplus16(ra_v[r, sl])
            b = _softplus16(rb_v[r, sl])
            s = a + b
            r0 = 1.0 / s
            r1 = r0 * (2.0 - s * r0)
            ra_v[r, sl] = a * r1
        return carry

    lax.fori_loop(0, BPW, body, 0, unroll=2)

    pltpu.sync_copy(ra_v, out_hbm.at[pl.ds(wid * BPW, BPW)])


@functools.cache
def _get_hbb_sc():
    return functools.partial(
        pl.kernel,
        out_type=jax.ShapeDtypeStruct((B, K), jnp.float32),
        mesh=_get_mesh(),
        compiler_params=pltpu.CompilerParams(use_tc_tiling_on_sc=False),
        scratch_types=[
            pltpu.VMEM((NCH, CHUNK), jnp.int32),
            pltpu.VMEM((BPW, K), jnp.float32),
            pltpu.VMEM((BPW, K), jnp.float32),
            pltpu.SemaphoreType.DMA,
        ],
    )(_hbb_body)


def kernel(site_idx, q_a_site, q_b_site):
    idx = site_idx.astype(jnp.int32).reshape(B // CHUNK, CHUNK)
    return _get_hbb_sc()(idx, q_a_site, q_b_site)


# speedup vs baseline: 1.1989x; 1.1989x over previous
import functools

import jax
import jax.numpy as jnp
from jax import lax
from jax.experimental import pallas as pl
from jax.experimental.pallas import tpu as pltpu
from jax.experimental.pallas import tpu_sc as plsc

N_SITES = 100000
K = 64
B = 16384
NC, NS, L = 2, 16, 16
NW = NC * NS
BPW = B // NW
CHUNK = 128
NCH = BPW // CHUNK

_C0 = 3.4730087503653806e-06
_C1 = 0.9997931695179835
_C2 = -0.4969826978177418
_C3 = 0.3146040987042588
_C4 = -0.18880110398530178
_C5 = 0.0817383926631096
_C6 = -0.017210668570108372

_LOG2E = 1.4426950408889634
_MAGIC = 12582912.0
_E0 = 1.0000000523545023
_E1 = 0.6931272466723738
_E2 = 0.24022211306668645
_E3 = 0.05587568397212921
_E4 = 0.009670788947925736


def _exp2_accurate(z):
    n = (z + _MAGIC) - _MAGIC
    f = z - n
    p = _E3 + f * _E4
    p = _E2 + f * p
    p = _E1 + f * p
    p = _E0 + f * p
    n_i = n.astype(jnp.int32)
    scale = lax.bitcast_convert_type(
        lax.shift_left(n_i + 127, jnp.int32(23)), jnp.float32
    )
    return p * scale


_D0 = 0.0009223163497825149
_D1 = 0.9797691943591391
_D2 = -0.3935581873890316
_D3 = 0.10669243657177084


def _softplus16(x):
    t = jnp.exp(-jnp.abs(x))
    p = _D2 + t * _D3
    p = _D1 + t * p
    p = _D0 + t * p
    return jnp.maximum(x, 0.0) + p


@functools.cache
def _get_mesh():
    return plsc.VectorSubcoreMesh(
        core_axis_name="c", subcore_axis_name="s", num_cores=NC, num_subcores=NS
    )


def _hbb_body(idx_hbm, qa_hbm, qb_hbm, out_hbm, idx_v, ra_v, rb_v, sem):
    wid = lax.axis_index("s") * NC + lax.axis_index("c")
    base = wid * NCH

    pltpu.sync_copy(idx_hbm.at[pl.ds(base, NCH)], idx_v)

    copies = []
    for j in range(NCH):
        copies.append(
            pltpu.async_copy(
                qa_hbm.at[idx_v.at[j]], ra_v.at[pl.ds(j * CHUNK, CHUNK)], sem
            )
        )
        copies.append(
            pltpu.async_copy(
                qb_hbm.at[idx_v.at[j]], rb_v.at[pl.ds(j * CHUNK, CHUNK)], sem
            )
        )
    for c in copies:
        c.wait()

    def body(r, carry):
        for c in range(K // L):
            sl = pl.ds(c * L, L)
            a = _softplus16(ra_v[r, sl])
            b = _softplus16(rb_v[r, sl])
            ra_v[r, sl] = a / (a + b)
        return carry

    lax.fori_loop(0, BPW, body, 0, unroll=4)

    pltpu.sync_copy(ra_v, out_hbm.at[pl.ds(wid * BPW, BPW)])


@functools.cache
def _get_hbb_sc():
    return functools.partial(
        pl.kernel,
        out_type=jax.ShapeDtypeStruct((B, K), jnp.float32),
        mesh=_get_mesh(),
        compiler_params=pltpu.CompilerParams(use_tc_tiling_on_sc=False),
        scratch_types=[
            pltpu.VMEM((NCH, CHUNK), jnp.int32),
            pltpu.VMEM((BPW, K), jnp.float32),
            pltpu.VMEM((BPW, K), jnp.float32),
            pltpu.SemaphoreType.DMA,
        ],
    )(_hbb_body)


def kernel(site_idx, q_a_site, q_b_site):
    idx = site_idx.astype(jnp.int32).reshape(B // CHUNK, CHUNK)
    return _get_hbb_sc()(idx, q_a_site, q_b_site)


# speedup vs baseline: 1.5323x; 1.2781x over previous
import functools

import jax
import jax.numpy as jnp
from jax import lax
from jax.experimental import pallas as pl
from jax.experimental.pallas import tpu as pltpu
from jax.experimental.pallas import tpu_sc as plsc

N_SITES = 100000
K = 64
B = 16384
NC, NS, L = 2, 16, 16
NW = NC * NS
BPW = B // NW
CHUNK = 128
NCH = BPW // CHUNK

_D0 = 0.0009223163497825149
_D1 = 0.9797691943591391
_D2 = -0.3935581873890316
_D3 = 0.10669243657177084


def _softplus16(x):
    t = jnp.exp(-jnp.abs(x))
    p = _D2 + t * _D3
    p = _D1 + t * p
    p = _D0 + t * p
    return jnp.maximum(x, 0.0) + p


@functools.cache
def _get_mesh():
    return plsc.VectorSubcoreMesh(
        core_axis_name="c", subcore_axis_name="s", num_cores=NC, num_subcores=NS
    )


def _hbb_body(idx_hbm, qa_hbm, qb_hbm, out_hbm, idx_v, ra_v, rb_v, gsems, wsem):
    wid = lax.axis_index("s") * NC + lax.axis_index("c")
    base = wid * NCH

    pltpu.sync_copy(idx_hbm.at[pl.ds(base, NCH)], idx_v)

    gathers = []
    for j in range(NCH):
        dst = pl.ds(j * CHUNK, CHUNK)
        gathers.append(
            (
                pltpu.async_copy(qa_hbm.at[idx_v.at[j]], ra_v.at[dst], gsems.at[j]),
                pltpu.async_copy(qb_hbm.at[idx_v.at[j]], rb_v.at[dst], gsems.at[j]),
            )
        )

    writes = []
    for j in range(NCH):
        for c in gathers[j]:
            c.wait()

        @plsc.parallel_loop(j * CHUNK, (j + 1) * CHUNK, unroll=2)
        def _(r):
            for c in range(K // L):
                sl = pl.ds(c * L, L)
                a = _softplus16(ra_v[r, sl])
                b = _softplus16(rb_v[r, sl])
                ra_v[r, sl] = a / (a + b)

        src = pl.ds(j * CHUNK, CHUNK)
        writes.append(
            pltpu.async_copy(
                ra_v.at[src], out_hbm.at[pl.ds(wid * BPW + j * CHUNK, CHUNK)], wsem
            )
        )

    for w in writes:
        w.wait()


@functools.cache
def _get_hbb_sc():
    return functools.partial(
        pl.kernel,
        out_type=jax.ShapeDtypeStruct((B, K), jnp.float32),
        mesh=_get_mesh(),
        compiler_params=pltpu.CompilerParams(use_tc_tiling_on_sc=False),
        scratch_types=[
            pltpu.VMEM((NCH, CHUNK), jnp.int32),
            pltpu.VMEM((BPW, K), jnp.float32),
            pltpu.VMEM((BPW, K), jnp.float32),
            pltpu.SemaphoreType.DMA((NCH,)),
            pltpu.SemaphoreType.DMA,
        ],
    )(_hbb_body)


def kernel(site_idx, q_a_site, q_b_site):
    idx = site_idx.astype(jnp.int32).reshape(B // CHUNK, CHUNK)
    return _get_hbb_sc()(idx, q_a_site, q_b_site)


# speedup vs baseline: 1.8357x; 1.1980x over previous
import functools

import jax
import jax.numpy as jnp
from jax import lax
from jax.experimental import pallas as pl
from jax.experimental.pallas import tpu as pltpu
from jax.experimental.pallas import tpu_sc as plsc

N_SITES = 100000
K = 64
B = 16384
NC, NS, L = 2, 16, 16
NW = NC * NS
BPW = B // NW
CHUNK = 128
NCH = BPW // CHUNK

_D0 = 0.0009223163497825149
_D1 = 0.9797691943591391
_D2 = -0.3935581873890316
_D3 = 0.10669243657177084


def _softplus16(x):
    t = jnp.exp(-jnp.abs(x))
    p = _D2 + t * _D3
    p = _D1 + t * p
    p = _D0 + t * p
    return jnp.maximum(x, 0.0) + p


@functools.cache
def _get_mesh():
    return plsc.VectorSubcoreMesh(
        core_axis_name="c", subcore_axis_name="s", num_cores=NC, num_subcores=NS
    )


def _hbb_body(idx_hbm, qab_hbm, out_hbm, idx_v, rows_v, gsems, wsem):
    wid = lax.axis_index("s") * NC + lax.axis_index("c")

    pltpu.sync_copy(idx_hbm.at[wid], idx_v)

    gathers = []
    for j in range(NCH):
        gathers.append(
            pltpu.async_copy(
                qab_hbm.at[idx_v.at[j]],
                rows_v.at[pl.ds(j * CHUNK, CHUNK)],
                gsems.at[j],
            )
        )

    writes = []
    for j in range(NCH):
        gathers[j].wait()

        @plsc.parallel_loop(j * CHUNK, (j + 1) * CHUNK, unroll=2)
        def _(r):
            for c in range(K // L):
                a = _softplus16(rows_v[r, pl.ds(c * L, L)])
                b = _softplus16(rows_v[r, pl.ds(K + c * L, L)])
                rows_v[r, pl.ds(c * L, L)] = a / (a + b)

        sl = pl.ds(j * CHUNK, CHUNK)
        writes.append(
            pltpu.async_copy(
                rows_v.at[sl], out_hbm.at[pl.ds(wid * BPW + j * CHUNK, CHUNK)], wsem
            )
        )

    for wr in writes:
        wr.wait()


@functools.cache
def _get_hbb_sc():
    return functools.partial(
        pl.kernel,
        out_type=jax.ShapeDtypeStruct((B, 2 * K), jnp.float32),
        mesh=_get_mesh(),
        scratch_types=[
            pltpu.VMEM((NCH, CHUNK), jnp.int32),
            pltpu.VMEM((BPW, 2 * K), jnp.float32),
            pltpu.SemaphoreType.DMA((NCH,)),
            pltpu.SemaphoreType.DMA,
        ],
    )(_hbb_body)


def kernel(site_idx, q_a_site, q_b_site):
    qab = jnp.concatenate([q_a_site, q_b_site], axis=1)
    idx = site_idx.astype(jnp.int32).reshape(NW, NCH, CHUNK)
    out = _get_hbb_sc()(idx, qab)
    return out[:, :K]
